# baseline (speedup 1.0000x reference)
import functools

import jax
import jax.numpy as jnp
from jax import lax
from jax.experimental import pallas as pl
from jax.experimental.pallas import tpu as pltpu
from jax.experimental.pallas import tpu_sc as plsc

E_DIM = 64
STREAM = 128
SUP = 512
K = SUP // STREAM
NBUF = 2


@functools.cache
def _build(B: int):
    info = plsc.get_sparse_core_info()
    nw = info.num_cores * info.num_subcores
    b_w = B // nw
    n_sup = b_w // SUP
    mesh = plsc.VectorSubcoreMesh(core_axis_name="c", subcore_axis_name="s")

    @functools.partial(
        pl.kernel,
        out_type=jax.ShapeDtypeStruct((B, E_DIM), jnp.float32),
        mesh=mesh,
        scratch_types=[
            pltpu.VMEM((NBUF, K, STREAM), jnp.int32),
            pltpu.VMEM((NBUF, K, STREAM), jnp.int32),
            pltpu.VMEM((NBUF, SUP, E_DIM), jnp.float32),
            pltpu.SemaphoreType.DMA((NBUF,)),
            pltpu.SemaphoreType.DMA((NBUF,)),
        ],
        compiler_params=pltpu.CompilerParams(
            use_tc_tiling_on_sc=False,
            needs_layout_passes=False,
            disable_bounds_checks=True,
        ),
    )
    def emb(x2_hbm, ev_hbm, w_hbm, out_hbm, gidx_v, sidx_v, rows_v, gsem, ssem):
        wid = lax.axis_index("s") * info.num_cores + lax.axis_index("c")
        base = wid * b_w

        def issue(b, g):
            row0 = (base + g * SUP) // STREAM
            pltpu.sync_copy(x2_hbm.at[pl.ds(row0, K)], gidx_v.at[b])
            pltpu.sync_copy(ev_hbm.at[pl.ds(row0, K)], sidx_v.at[b])
            for k in range(K):
                pltpu.async_copy(
                    w_hbm.at[gidx_v.at[b, k]],
                    rows_v.at[b, pl.ds(k * STREAM, STREAM)],
                    gsem.at[b],
                )

        def wait_gathers(b):
            pltpu.make_async_copy(
                w_hbm.at[pl.ds(0, SUP)], rows_v.at[b], gsem.at[b]
            ).wait()

        def scatter(b):
            for k in range(K):
                pltpu.async_copy(
                    rows_v.at[b, pl.ds(k * STREAM, STREAM)],
                    out_hbm.at[sidx_v.at[b, k]],
                    ssem.at[b],
                )

        def wait_scatters(b):
            pltpu.make_async_copy(
                rows_v.at[b], out_hbm.at[pl.ds(0, SUP)], ssem.at[b]
            ).wait()

        issue(0, 0)

        @pl.loop(0, n_sup, step=NBUF)
        def _outer(g0):
            for b in range(NBUF):
                g = g0 + b
                nb = (b + 1) % NBUF

                @pl.when(g + 1 < n_sup)
                def _prefetch():
                    @pl.when(g >= 1)
                    def _reuse_guard():
                        wait_scatters(nb)

                    issue(nb, g + 1)

                wait_gathers(b)
                scatter(b)

        wait_scatters(0)
        wait_scatters(1)

    return emb


@jax.jit
def kernel(X, W):
    batch, seq = X.shape
    B = batch * seq
    x2 = (X.reshape(B).astype(jnp.int32) * 2).reshape(B // STREAM, STREAM)
    ev = (jnp.arange(B, dtype=jnp.int32) % 819199).reshape(B // STREAM, STREAM)
    out = _build(B)(x2, ev, W)
    return out.reshape(batch, seq, E_DIM)


# speedup vs baseline: 1.2695x; 1.2695x over previous
import functools

import jax
import jax.numpy as jnp
from jax import lax
from jax.experimental import pallas as pl
from jax.experimental.pallas import tpu as pltpu
from jax.experimental.pallas import tpu_sc as plsc
from jax.experimental.layout import Layout, with_layout_constraint

E_DIM = 64
STREAM = 128
SUP = 512
K = SUP // STREAM
NBUF = 2


@functools.cache
def _build(B: int):
    info = plsc.get_sparse_core_info()
    nw = info.num_cores * info.num_subcores
    b_w = B // nw
    n_sup = b_w // SUP
    mesh = plsc.VectorSubcoreMesh(core_axis_name="c", subcore_axis_name="s")

    @functools.partial(
        pl.kernel,
        out_type=jax.ShapeDtypeStruct((B, E_DIM), jnp.float32),
        mesh=mesh,
        scratch_types=[
            pltpu.VMEM((NBUF, K, STREAM), jnp.int32),
            pltpu.VMEM((NBUF, SUP, E_DIM), jnp.float32),
            pltpu.SemaphoreType.DMA((NBUF,)),
        ],
        compiler_params=pltpu.CompilerParams(use_tc_tiling_on_sc=False),
    )
    def emb(x_hbm, w_hbm, out_hbm, idx_v, rows_v, gsem):
        wid = lax.axis_index("s") * info.num_cores + lax.axis_index("c")
        base = wid * b_w

        def issue(b, g):
            row0 = (base + g * SUP) // STREAM
            pltpu.sync_copy(x_hbm.at[pl.ds(row0, K)], idx_v.at[b])
            for k in range(K):
                pltpu.async_copy(
                    w_hbm.at[idx_v.at[b, k]],
                    rows_v.at[b, pl.ds(k * STREAM, STREAM)],
                    gsem.at[b],
                )

        def wait_gathers(b):
            pltpu.make_async_copy(
                w_hbm.at[pl.ds(0, SUP)], rows_v.at[b], gsem.at[b]
            ).wait()

        issue(0, 0)

        @pl.loop(0, n_sup, step=NBUF)
        def _outer(g0):
            for b in range(NBUF):
                g = g0 + b
                nb = (b + 1) % NBUF

                @pl.when(g + 1 < n_sup)
                def _prefetch():
                    issue(nb, g + 1)

                wait_gathers(b)
                pltpu.sync_copy(
                    rows_v.at[b], out_hbm.at[pl.ds(base + g * SUP, SUP)]
                )

    return emb


@jax.jit
def kernel(X, W):
    batch, seq = X.shape
    B = batch * seq
    flat_idx = (X.reshape(B).astype(jnp.int32) * 2).reshape(B // STREAM, STREAM)
    Wc = with_layout_constraint(
        W, Layout(major_to_minor=(0, 1), tiling=((8,),))
    )
    out = _build(B)(flat_idx, Wc)
    return out.reshape(batch, seq, E_DIM)


# speedup vs baseline: 1.2728x; 1.0026x over previous
import functools

import jax
import jax.numpy as jnp
from jax import lax
from jax.experimental import pallas as pl
from jax.experimental.pallas import tpu as pltpu
from jax.experimental.pallas import tpu_sc as plsc
from jax.experimental.layout import Layout, with_layout_constraint

E_DIM = 64
STREAM = 128
SUP = 512
K = SUP // STREAM
NBUF = 2


@functools.cache
def _build(B: int):
    info = plsc.get_sparse_core_info()
    nw = info.num_cores * info.num_subcores
    b_w = B // nw
    n_sup = b_w // SUP
    mesh = plsc.VectorSubcoreMesh(core_axis_name="c", subcore_axis_name="s")

    @functools.partial(
        pl.kernel,
        out_type=jax.ShapeDtypeStruct((B, E_DIM), jnp.float32),
        mesh=mesh,
        scratch_types=[
            pltpu.VMEM((NBUF, K, STREAM), jnp.int32),
            pltpu.VMEM((NBUF, SUP, E_DIM), jnp.float32),
            pltpu.SemaphoreType.DMA((NBUF,)),
        ],
        compiler_params=pltpu.CompilerParams(use_tc_tiling_on_sc=False),
    )
    def emb(x_hbm, w_hbm, out_hbm, idx_v, rows_v, gsem):
        wid = lax.axis_index("s") * info.num_cores + lax.axis_index("c")
        base = wid * b_w

        def issue(b, g):
            row0 = (base + g * SUP) // STREAM
            pltpu.sync_copy(x_hbm.at[pl.ds(row0, K)], idx_v.at[b])
            for k in range(K):
                pltpu.async_copy(
                    w_hbm.at[idx_v.at[b, k]],
                    rows_v.at[b, pl.ds(k * STREAM, STREAM)],
                    gsem.at[b],
                )

        def wait_gathers(b):
            pltpu.make_async_copy(
                w_hbm.at[pl.ds(0, SUP)], rows_v.at[b], gsem.at[b]
            ).wait()

        issue(0, 0)

        @pl.loop(0, n_sup, step=NBUF)
        def _outer(g0):
            for b in range(NBUF):
                g = g0 + b
                nb = (b + 1) % NBUF

                @pl.when(g + 1 < n_sup)
                def _prefetch():
                    issue(nb, g + 1)

                wait_gathers(b)
                pltpu.sync_copy(
                    rows_v.at[b], out_hbm.at[pl.ds(base + g * SUP, SUP)]
                )

    return emb


@jax.jit
def kernel(X, W):
    batch, seq = X.shape
    B = batch * seq
    flat_idx = (X.reshape(B).astype(jnp.int32) * 2).reshape(B // STREAM, STREAM)
    Wc = with_layout_constraint(
        W, Layout(major_to_minor=(0, 1), tiling=((8,),))
    )
    out = _build(B)(flat_idx, Wc)
    out3 = out.reshape(batch, seq, E_DIM)
    return with_layout_constraint(
        out3, Layout(major_to_minor=(1, 2, 0))
    )


# speedup vs baseline: 1.7879x; 1.4047x over previous
import functools

import jax
import jax.numpy as jnp
from jax import lax
from jax.experimental import pallas as pl
from jax.experimental.pallas import tpu as pltpu
from jax.experimental.pallas import tpu_sc as plsc
from jax.experimental.layout import Layout, with_layout_constraint

E_DIM = 64
ROW_PAD = 128
STREAM = 128
SUP = 256
K = SUP // STREAM
NBUF = 2


@functools.cache
def _build(B: int):
    info = plsc.get_sparse_core_info()
    nw = info.num_cores * info.num_subcores
    b_w = B // nw
    n_sup = b_w // SUP
    mesh = plsc.VectorSubcoreMesh(core_axis_name="c", subcore_axis_name="s")

    @functools.partial(
        pl.kernel,
        out_type=jax.ShapeDtypeStruct((B, ROW_PAD), jnp.float32),
        mesh=mesh,
        scratch_types=[
            pltpu.VMEM((NBUF, K, STREAM), jnp.int32),
            pltpu.VMEM((NBUF, SUP, E_DIM), jnp.float32),
            pltpu.SemaphoreType.DMA((NBUF,)),
        ],
        compiler_params=pltpu.CompilerParams(use_tc_tiling_on_sc=False),
    )
    def emb(x_hbm, w_hbm, out_hbm, idx_v, rows_v, gsem):
        wid = lax.axis_index("s") * info.num_cores + lax.axis_index("c")
        base = wid * b_w

        def issue(b, g):
            row0 = (base + g * SUP) // STREAM
            pltpu.sync_copy(x_hbm.at[pl.ds(row0, K)], idx_v.at[b])
            for k in range(K):
                pltpu.async_copy(
                    w_hbm.at[idx_v.at[b, k]],
                    rows_v.at[b, pl.ds(k * STREAM, STREAM)],
                    gsem.at[b],
                )

        def wait_gathers(b):
            pltpu.make_async_copy(
                w_hbm.at[pl.ds(0, SUP)], rows_v.at[b], gsem.at[b]
            ).wait()

        issue(0, 0)

        @pl.loop(0, n_sup, step=NBUF)
        def _outer(g0):
            for b in range(NBUF):
                g = g0 + b
                nb = (b + 1) % NBUF

                @pl.when(g + 1 < n_sup)
                def _prefetch():
                    issue(nb, g + 1)

                wait_gathers(b)
                pltpu.sync_copy(
                    rows_v.at[b],
                    out_hbm.at[pl.ds(base + g * SUP, SUP), pl.ds(0, E_DIM)],
                )

    return emb


@jax.jit
def kernel(X, W):
    batch, seq = X.shape
    B = batch * seq
    flat_idx = (X.reshape(B).astype(jnp.int32) * 2).reshape(B // STREAM, STREAM)
    Wc = with_layout_constraint(
        W, Layout(major_to_minor=(0, 1), tiling=((8, 128),))
    )
    out = _build(B)(flat_idx, Wc)
    out3 = out[:, :E_DIM].reshape(batch, seq, E_DIM)
    return with_layout_constraint(out3, Layout(major_to_minor=(1, 2, 0)))
